# baseline (speedup 1.0000x reference)
import jax
import jax.numpy as jnp
import numpy as np
from jax import lax
from jax.experimental import pallas as pl
from jax.experimental.pallas import tpu as pltpu
from jax.experimental.pallas import tpu_sc as plsc


def kernel(frames):
    C, T, H, W = frames.shape
    S = T // 4

    idx = np.linspace(0.0, T - 1, S).astype(np.int64)
    jt = (np.arange(T) * (S - 1) + (T - 2)) // (T - 1)
    assert np.array_equal(jt, np.searchsorted(idx, np.arange(T)))
    assert np.all(np.diff(idx) > 0)

    N = C * T
    NC, NS = 2, 16
    NW = NC * NS
    assert N % NW == 0
    RPW = N // NW

    mesh = plsc.VectorSubcoreMesh(
        core_axis_name="c", subcore_axis_name="s")

    def body(x_hbm, fast_hbm, slow_hbm, fsem, ssem):
        wid = lax.axis_index("s") * NC + lax.axis_index("c")

        def info(i):
            r = wid * RPW + i
            t = lax.rem(r, T)
            ch = lax.div(r, T)
            j0 = (t * (S - 1) + (T - 2)) // (T - 1)
            j1 = ((t + 1) * (S - 1) + (T - 2)) // (T - 1)
            issel = jnp.logical_or(t == T - 1, j1 != j0)
            return ch, t, j0, issel

        def fast_cp(i):
            ch, t, _, _ = info(i)
            return pltpu.make_async_copy(
                x_hbm.at[ch, t], fast_hbm.at[ch, t], fsem)

        def slow_cp(i):
            ch, t, j0, _ = info(i)
            return pltpu.make_async_copy(
                x_hbm.at[ch, t], slow_hbm.at[ch, j0], ssem)

        for i in range(RPW):
            fast_cp(i).start()
            _, _, _, issel = info(i)

            @pl.when(issel)
            def _():
                slow_cp(i).start()

        for i in range(RPW):
            fast_cp(i).wait()
            _, _, _, issel = info(i)

            @pl.when(issel)
            def _():
                slow_cp(i).wait()

    run = pl.kernel(
        body,
        out_type=[
            jax.ShapeDtypeStruct((C, T, H, W), frames.dtype),
            jax.ShapeDtypeStruct((C, S, H, W), frames.dtype),
        ],
        mesh=mesh,
        scratch_types=[
            pltpu.SemaphoreType.DMA,
            pltpu.SemaphoreType.DMA,
        ],
    )
    fast, slow = run(frames)
    return (slow, fast)


# speedup vs baseline: 36.4028x; 36.4028x over previous
import jax
import jax.numpy as jnp
import numpy as np
from jax import lax
from jax.experimental import pallas as pl
from jax.experimental.pallas import tpu as pltpu
from jax.experimental.pallas import tpu_sc as plsc


def kernel(frames):
    C, T, H, W = frames.shape
    S = T // 4

    idx = np.linspace(0.0, T - 1, S).astype(np.int64)
    jt = (np.arange(T) * (S - 1) + (T - 2)) // (T - 1)
    assert np.array_equal(jt, np.searchsorted(idx, np.arange(T)))
    assert np.all(np.diff(idx) > 0)

    N = C * T
    NC, NS = 2, 16
    NW = NC * NS
    assert N % NW == 0
    RPW = N // NW

    mesh = plsc.VectorSubcoreMesh(
        core_axis_name="c", subcore_axis_name="s")

    HC = H // 2
    assert H % 2 == 0 and HC % 8 == 0
    NCH = RPW * 2

    def body(x_hbm, fast_hbm, slow_hbm, buf, insem, outsem):
        wid = lax.axis_index("s") * NC + lax.axis_index("c")

        def info(k):
            r = wid * RPW + (k // 2)
            t = lax.rem(r, T)
            ch = lax.div(r, T)
            j0 = (t * (S - 1) + (T - 2)) // (T - 1)
            j1 = ((t + 1) * (S - 1) + (T - 2)) // (T - 1)
            issel = jnp.logical_or(t == T - 1, j1 != j0)
            h0 = (k % 2) * HC
            return ch, t, j0, issel, h0

        def in_cp(k, s):
            ch, t, _, _, h0 = info(k)
            return pltpu.make_async_copy(
                x_hbm.at[ch, t, pl.ds(h0, HC)], buf.at[s], insem.at[s])

        def fast_cp(k, s):
            ch, t, _, _, h0 = info(k)
            return pltpu.make_async_copy(
                buf.at[s], fast_hbm.at[ch, t, pl.ds(h0, HC)], outsem.at[s])

        def slow_cp(k, s):
            ch, _, j0, _, h0 = info(k)
            return pltpu.make_async_copy(
                buf.at[s], slow_hbm.at[ch, j0, pl.ds(h0, HC)], outsem.at[s])

        def out_wait(k, s):
            fast_cp(k, s).wait()
            _, _, _, issel, _ = info(k)

            @pl.when(issel)
            def _():
                slow_cp(k, s).wait()

        in_cp(0, 0).start()
        for k in range(NCH):
            s = k % 2
            in_cp(k, s).wait()
            fast_cp(k, s).start()
            _, _, _, issel, _ = info(k)

            @pl.when(issel)
            def _():
                slow_cp(k, s).start()

            if k + 1 < NCH:
                if k >= 1:
                    out_wait(k - 1, (k - 1) % 2)
                in_cp(k + 1, (k + 1) % 2).start()
        for k in range(max(NCH - 2, 0), NCH):
            out_wait(k, k % 2)

    run = pl.kernel(
        body,
        out_type=[
            jax.ShapeDtypeStruct((C, T, H, W), frames.dtype),
            jax.ShapeDtypeStruct((C, S, H, W), frames.dtype),
        ],
        mesh=mesh,
        scratch_types=[
            pltpu.VMEM((2, HC, W), frames.dtype),
            pltpu.SemaphoreType.DMA((2,)),
            pltpu.SemaphoreType.DMA((2,)),
        ],
    )
    fast, slow = run(frames)
    return (slow, fast)
